# Initial kernel scaffold; baseline (speedup 1.0000x reference)
import jax
import jax.numpy as jnp
from jax.experimental import pallas as pl


def kernel(x_u, x_v, edge_u, edge_v, edge_val, edge_val_t, W):
    raise NotImplementedError("write your pallas kernel here")


# speedup vs baseline: 6.6724x; 6.6724x over previous
import jax
import jax.numpy as jnp
from jax import lax
from jax.experimental import pallas as pl
from jax.experimental.pallas import tpu as pltpu
from jax.experimental.pallas import tpu_sc as plsc

N_NODES = 25000
D_IN = 128
D_OUT = 128
NSUP = 4
DS = D_OUT // NSUP
E = 160000
N_TILES = 16
CHUNK = 128
CHUNKS = 80
E_PAD = N_TILES * CHUNKS * CHUNK
ACC_ROWS = 25088
TILE_ROWS = ACC_ROWS // N_TILES
RB = 224
RB_ITERS = TILE_ROWS // RB


def _project_body(xu_ref, xv_ref, w_ref, *out_refs):
    w = w_ref[...]
    hu = jnp.dot(xu_ref[...], w, preferred_element_type=jnp.float32)
    hv = jnp.dot(xv_ref[...], w, preferred_element_type=jnp.float32)
    for i in range(NSUP):
        out_refs[i][...] = hu[:, i * DS:(i + 1) * DS]
        out_refs[NSUP + i][...] = hv[:, i * DS:(i + 1) * DS]


def _project(x_u, x_v, W):
    rb = 1000
    return pl.pallas_call(
        _project_body,
        grid=(N_NODES // rb,),
        in_specs=[
            pl.BlockSpec((rb, D_IN), lambda r: (r, 0)),
            pl.BlockSpec((rb, D_IN), lambda r: (r, 0)),
            pl.BlockSpec((D_IN, D_OUT), lambda r: (0, 0)),
        ],
        out_specs=[pl.BlockSpec((rb, DS), lambda r: (r, 0))] * (2 * NSUP),
        out_shape=[jax.ShapeDtypeStruct((N_NODES, DS), jnp.float32)] * (2 * NSUP),
    )(x_u, x_v, W)


def _sc_body(tu0, tu1, tu2, tu3, tv0, tv1, tv2, tv3,
             eu, ev, val, valt, out_u, out_v,
             acc, zbuf, rows, rbuf, isrc, idst, vbuf):
    c = lax.axis_index("c")
    s = lax.axis_index("s")
    t_u = [tu0, tu1, tu2, tu3]
    t_v = [tv0, tv1, tv2, tv3]
    base = s * TILE_ROWS

    @pl.loop(0, RB)
    def _zinit(r):
        for h in range(2):
            zbuf[r, pl.ds(16 * h, 16)] = jnp.zeros((16,), jnp.float32)

    def load_phase(e_src, e_dst, e_val):
        for k in range(RB_ITERS):
            pltpu.sync_copy(zbuf, acc.at[pl.ds(base + k * RB, RB)])
        pltpu.sync_copy(e_src, isrc)
        pltpu.sync_copy(e_dst, idst)
        pltpu.sync_copy(e_val, vbuf)

    def edge_phase(tbl):
        @pl.loop(0, CHUNKS)
        def _chunk(j):
            pltpu.sync_copy(tbl.at[isrc.at[j]], rows)

            @pl.loop(0, CHUNK // 16)
            def _scale(g):
                vv = vbuf[j, pl.ds(g * 16, 16)]
                for t in range(16):
                    e = g * 16 + t
                    v = vv[t]
                    for h in range(2):
                        sl = pl.ds(16 * h, 16)
                        rows[e, sl] = rows[e, sl] * v

            pltpu.sync_copy(rows, acc.at[idst.at[j]], add=True)

    def store_phase(out_ref, col):
        for k in range(RB_ITERS):
            r0 = base + k * RB
            pltpu.sync_copy(acc.at[pl.ds(r0, RB)], rbuf)

            @pl.loop(0, RB)
            def _relu(r):
                for h in range(2):
                    sl = pl.ds(16 * h, 16)
                    rbuf[r, sl] = jnp.maximum(rbuf[r, sl], 0.0)

            pltpu.sync_copy(rbuf, out_ref.at[pl.ds(r0, RB), pl.ds(col, DS)])

    for p in range(NSUP):
        @pl.when(c == 0)
        def _():
            load_phase(ev.at[p, s], eu.at[p, s], val.at[p, s])

        @pl.when(c == 1)
        def _():
            load_phase(eu.at[p, s], ev.at[p, s], valt.at[p, s])

        plsc.subcore_barrier()

        @pl.when(c == 0)
        def _():
            edge_phase(t_v[p])

        @pl.when(c == 1)
        def _():
            edge_phase(t_u[p])

        plsc.subcore_barrier()

        @pl.when(c == 0)
        def _():
            store_phase(out_u, p * DS)

        @pl.when(c == 1)
        def _():
            store_phase(out_v, p * DS)


_SC_CALL_CACHE = []


def _sc_call(*args):
    if not _SC_CALL_CACHE:
        _SC_CALL_CACHE.append(pl.kernel(
            _sc_body,
            out_type=[jax.ShapeDtypeStruct((ACC_ROWS, D_OUT), jnp.float32)] * 2,
            mesh=plsc.VectorSubcoreMesh(core_axis_name="c", subcore_axis_name="s"),
            compiler_params=pltpu.CompilerParams(use_tc_tiling_on_sc=False),
            scratch_types=[
                pltpu.VMEM_SHARED((ACC_ROWS, DS), jnp.float32),
                pltpu.VMEM((RB, DS), jnp.float32),
                pltpu.VMEM((CHUNK, DS), jnp.float32),
                pltpu.VMEM((RB, DS), jnp.float32),
                pltpu.VMEM((CHUNKS, CHUNK), jnp.int32),
                pltpu.VMEM((CHUNKS, CHUNK), jnp.int32),
                pltpu.VMEM((CHUNKS, CHUNK), jnp.float32),
            ],
        ))
    return _SC_CALL_CACHE[0](*args)


def kernel(x_u, x_v, edge_u, edge_v, edge_val, edge_val_t, W):
    tabs = _project(x_u, x_v, W)
    pad = E_PAD - E

    def pad4(a):
        return jnp.pad(a, ((0, 0), (0, pad))).reshape(NSUP, N_TILES, CHUNKS, CHUNK)

    eu = pad4(edge_u)
    ev = pad4(edge_v)
    val = pad4(edge_val)
    valt = pad4(edge_val_t)
    out_u, out_v = _sc_call(*tabs, eu, ev, val, valt)
    return out_u[:N_NODES], out_v[:N_NODES]


# speedup vs baseline: 8.9260x; 1.3377x over previous
import jax
import jax.numpy as jnp
from jax import lax
from jax.experimental import pallas as pl
from jax.experimental.pallas import tpu as pltpu
from jax.experimental.pallas import tpu_sc as plsc

N_NODES = 25000
D_IN = 128
D_OUT = 128
NSUP = 4
DS = D_OUT // NSUP
E = 160000
N_TILES = 16
CHUNK = 128
CHUNKS = 80
E_PAD = N_TILES * CHUNKS * CHUNK
ACC_ROWS = 25088
TILE_ROWS = ACC_ROWS // N_TILES
RB = 224
RB_ITERS = TILE_ROWS // RB


def _project_body(xu_ref, xv_ref, w_ref, *out_refs):
    w = w_ref[...]
    hu = jnp.dot(xu_ref[...], w, preferred_element_type=jnp.float32)
    hv = jnp.dot(xv_ref[...], w, preferred_element_type=jnp.float32)
    for i in range(NSUP):
        out_refs[i][...] = hu[:, i * DS:(i + 1) * DS]
        out_refs[NSUP + i][...] = hv[:, i * DS:(i + 1) * DS]


def _project(x_u, x_v, W):
    rb = 1000
    return pl.pallas_call(
        _project_body,
        grid=(N_NODES // rb,),
        in_specs=[
            pl.BlockSpec((rb, D_IN), lambda r: (r, 0)),
            pl.BlockSpec((rb, D_IN), lambda r: (r, 0)),
            pl.BlockSpec((D_IN, D_OUT), lambda r: (0, 0)),
        ],
        out_specs=[pl.BlockSpec((rb, DS), lambda r: (r, 0))] * (2 * NSUP),
        out_shape=[jax.ShapeDtypeStruct((N_NODES, DS), jnp.float32)] * (2 * NSUP),
    )(x_u, x_v, W)


def _sc_body(tu0, tu1, tu2, tu3, tv0, tv1, tv2, tv3,
             eu, ev, val, valt, out_u, out_v,
             acc, zbuf, rows, rows_b, rbuf, isrc, idst, vbuf, gsem, gsem_b):
    c = lax.axis_index("c")
    s = lax.axis_index("s")
    t_u = [tu0, tu1, tu2, tu3]
    t_v = [tv0, tv1, tv2, tv3]
    base = s * TILE_ROWS

    @pl.loop(0, RB)
    def _zinit(r):
        for h in range(2):
            zbuf[r, pl.ds(16 * h, 16)] = jnp.zeros((16,), jnp.float32)

    def load_phase(e_src, e_dst, e_val):
        for k in range(RB_ITERS):
            pltpu.sync_copy(zbuf, acc.at[pl.ds(base + k * RB, RB)])
        pltpu.sync_copy(e_src, isrc)
        pltpu.sync_copy(e_dst, idst)
        pltpu.sync_copy(e_val, vbuf)

    def edge_phase(tbl):
        bufs = ((rows, gsem), (rows_b, gsem_b))
        for b, (buf, sem) in enumerate(bufs):
            pltpu.async_copy(tbl.at[isrc.at[b]], buf, sem)

        @pl.loop(0, CHUNKS, step=2)
        def _chunk(j):
            for b, (buf, sem) in enumerate(bufs):
                jj = j + b
                pltpu.make_async_copy(tbl.at[isrc.at[jj]], buf, sem).wait()

                @pl.loop(0, CHUNK // 16)
                def _scale(g):
                    vv = vbuf[jj, pl.ds(g * 16, 16)]
                    for t in range(16):
                        e = g * 16 + t
                        v = vv[t]
                        for h in range(2):
                            sl = pl.ds(16 * h, 16)
                            buf[e, sl] = buf[e, sl] * v

                pltpu.sync_copy(buf, acc.at[idst.at[jj]], add=True)

                @pl.when(jj + 2 < CHUNKS)
                def _():
                    pltpu.async_copy(tbl.at[isrc.at[jj + 2]], buf, sem)

    def store_phase(out_ref, col):
        for k in range(RB_ITERS):
            r0 = base + k * RB
            pltpu.sync_copy(acc.at[pl.ds(r0, RB)], rbuf)

            @pl.loop(0, RB)
            def _relu(r):
                for h in range(2):
                    sl = pl.ds(16 * h, 16)
                    rbuf[r, sl] = jnp.maximum(rbuf[r, sl], 0.0)

            pltpu.sync_copy(rbuf, out_ref.at[pl.ds(r0, RB), pl.ds(col, DS)])

    for p in range(NSUP):
        @pl.when(c == 0)
        def _():
            load_phase(ev.at[p, s], eu.at[p, s], val.at[p, s])

        @pl.when(c == 1)
        def _():
            load_phase(eu.at[p, s], ev.at[p, s], valt.at[p, s])

        plsc.subcore_barrier()

        @pl.when(c == 0)
        def _():
            edge_phase(t_v[p])

        @pl.when(c == 1)
        def _():
            edge_phase(t_u[p])

        plsc.subcore_barrier()

        @pl.when(c == 0)
        def _():
            store_phase(out_u, p * DS)

        @pl.when(c == 1)
        def _():
            store_phase(out_v, p * DS)


_SC_CALL_CACHE = []


def _sc_call(*args):
    if not _SC_CALL_CACHE:
        _SC_CALL_CACHE.append(pl.kernel(
            _sc_body,
            out_type=[jax.ShapeDtypeStruct((ACC_ROWS, D_OUT), jnp.float32)] * 2,
            mesh=plsc.VectorSubcoreMesh(core_axis_name="c", subcore_axis_name="s"),
            compiler_params=pltpu.CompilerParams(use_tc_tiling_on_sc=False),
            scratch_types=[
                pltpu.VMEM_SHARED((ACC_ROWS, DS), jnp.float32),
                pltpu.VMEM((RB, DS), jnp.float32),
                pltpu.VMEM((CHUNK, DS), jnp.float32),
                pltpu.VMEM((CHUNK, DS), jnp.float32),
                pltpu.VMEM((RB, DS), jnp.float32),
                pltpu.VMEM((CHUNKS, CHUNK), jnp.int32),
                pltpu.VMEM((CHUNKS, CHUNK), jnp.int32),
                pltpu.VMEM((CHUNKS, CHUNK), jnp.float32),
                pltpu.SemaphoreType.DMA,
                pltpu.SemaphoreType.DMA,
            ],
        ))
    return _SC_CALL_CACHE[0](*args)


def kernel(x_u, x_v, edge_u, edge_v, edge_val, edge_val_t, W):
    tabs = _project(x_u, x_v, W)
    pad = E_PAD - E

    def pad4(a):
        return jnp.pad(a, ((0, 0), (0, pad))).reshape(NSUP, N_TILES, CHUNKS, CHUNK)

    eu = pad4(edge_u)
    ev = pad4(edge_v)
    val = pad4(edge_val)
    valt = pad4(edge_val_t)
    out_u, out_v = _sc_call(*tabs, eu, ev, val, valt)
    return out_u[:N_NODES], out_v[:N_NODES]


# speedup vs baseline: 9.2319x; 1.0343x over previous
import jax
import jax.numpy as jnp
from jax import lax
from jax.experimental import pallas as pl
from jax.experimental.pallas import tpu as pltpu
from jax.experimental.pallas import tpu_sc as plsc

N_NODES = 25000
D_IN = 128
D_OUT = 128
NSUP = 4
DS = D_OUT // NSUP
E = 160000
N_TILES = 16
CHUNK = 128
CHUNKS = 80
E_PAD = N_TILES * CHUNKS * CHUNK
ACC_ROWS = 25088
TILE_ROWS = ACC_ROWS // N_TILES
RB = 224
RB_ITERS = TILE_ROWS // RB


def _project_body(xu_ref, xv_ref, w_ref, *out_refs):
    w = w_ref[...]
    hu = jnp.dot(xu_ref[...], w, preferred_element_type=jnp.float32)
    hv = jnp.dot(xv_ref[...], w, preferred_element_type=jnp.float32)
    for i in range(NSUP):
        out_refs[i][...] = hu[:, i * DS:(i + 1) * DS]
        out_refs[NSUP + i][...] = hv[:, i * DS:(i + 1) * DS]


def _project(x_u, x_v, W):
    rb = 1000
    return pl.pallas_call(
        _project_body,
        grid=(N_NODES // rb,),
        in_specs=[
            pl.BlockSpec((rb, D_IN), lambda r: (r, 0)),
            pl.BlockSpec((rb, D_IN), lambda r: (r, 0)),
            pl.BlockSpec((D_IN, D_OUT), lambda r: (0, 0)),
        ],
        out_specs=[pl.BlockSpec((rb, DS), lambda r: (r, 0))] * (2 * NSUP),
        out_shape=[jax.ShapeDtypeStruct((N_NODES, DS), jnp.float32)] * (2 * NSUP),
    )(x_u, x_v, W)


def _sc_body(tu0, tu1, tu2, tu3, tv0, tv1, tv2, tv3,
             eu, ev, val, valt, out_u, out_v,
             acc, zbuf, rows, rows_b, rows_c, rows_d, rbuf, isrc, idst, vbuf,
             gsem, gsem_b, gsem_c, gsem_d, ssem, ssem_b, ssem_c, ssem_d, lsem):
    c = lax.axis_index("c")
    s = lax.axis_index("s")
    t_u = [tu0, tu1, tu2, tu3]
    t_v = [tv0, tv1, tv2, tv3]
    base = s * TILE_ROWS

    @pl.loop(0, RB)
    def _zinit(r):
        for h in range(2):
            zbuf[r, pl.ds(16 * h, 16)] = jnp.zeros((16,), jnp.float32)

    def load_phase(e_src, e_dst, e_val):
        for k in range(RB_ITERS):
            pltpu.sync_copy(zbuf, acc.at[pl.ds(base + k * RB, RB)])
        pltpu.sync_copy(e_src, isrc)
        pltpu.sync_copy(e_dst, idst)
        pltpu.sync_copy(e_val, vbuf)

    def edge_phase(tbl):
        bufs = (rows, rows_b, rows_c, rows_d)
        gsems = (gsem, gsem_b, gsem_c, gsem_d)
        ssems = (ssem, ssem_b, ssem_c, ssem_d)
        for b in range(2):
            pltpu.async_copy(tbl.at[isrc.at[b]], bufs[b], gsems[b])

        @pl.loop(0, CHUNKS, step=4)
        def _chunk(j):
            for b in range(4):
                jj = j + b
                pltpu.make_async_copy(tbl.at[isrc.at[jj]], bufs[b], gsems[b]).wait()

                @pl.loop(0, CHUNK // 16)
                def _scale(g):
                    vv = vbuf[jj, pl.ds(g * 16, 16)]
                    for t in range(16):
                        e = g * 16 + t
                        v = vv[t]
                        for h in range(2):
                            sl = pl.ds(16 * h, 16)
                            bufs[b][e, sl] = bufs[b][e, sl] * v

                b2 = (b + 2) % 4

                @pl.when(jj + 2 < CHUNKS)
                def _():
                    pltpu.async_copy(tbl.at[isrc.at[jj + 2]], bufs[b2], gsems[b2])

                pltpu.sync_copy(bufs[b], acc.at[idst.at[jj]], add=True)

    def store_phase(out_ref, col):
        for k in range(RB_ITERS):
            r0 = base + k * RB
            pltpu.sync_copy(acc.at[pl.ds(r0, RB)], rbuf)

            @pl.loop(0, RB)
            def _relu(r):
                for h in range(2):
                    sl = pl.ds(16 * h, 16)
                    rbuf[r, sl] = jnp.maximum(rbuf[r, sl], 0.0)

            pltpu.sync_copy(rbuf, out_ref.at[pl.ds(r0, RB), pl.ds(col, DS)])

    for p in range(NSUP):
        @pl.when(c == 0)
        def _():
            load_phase(ev.at[p, s], eu.at[p, s], val.at[p, s])

        @pl.when(c == 1)
        def _():
            load_phase(eu.at[p, s], ev.at[p, s], valt.at[p, s])

        plsc.subcore_barrier()

        @pl.when(c == 0)
        def _():
            edge_phase(t_v[p])

        @pl.when(c == 1)
        def _():
            edge_phase(t_u[p])

        plsc.subcore_barrier()

        @pl.when(c == 0)
        def _():
            store_phase(out_u, p * DS)

        @pl.when(c == 1)
        def _():
            store_phase(out_v, p * DS)


_SC_CALL_CACHE = []


def _sc_call(*args):
    if not _SC_CALL_CACHE:
        _SC_CALL_CACHE.append(pl.kernel(
            _sc_body,
            out_type=[jax.ShapeDtypeStruct((ACC_ROWS, D_OUT), jnp.float32)] * 2,
            mesh=plsc.VectorSubcoreMesh(core_axis_name="c", subcore_axis_name="s"),
            compiler_params=pltpu.CompilerParams(use_tc_tiling_on_sc=False),
            scratch_types=[
                pltpu.VMEM_SHARED((ACC_ROWS, DS), jnp.float32),
                pltpu.VMEM((RB, DS), jnp.float32),
                pltpu.VMEM((CHUNK, DS), jnp.float32),
                pltpu.VMEM((CHUNK, DS), jnp.float32),
                pltpu.VMEM((CHUNK, DS), jnp.float32),
                pltpu.VMEM((CHUNK, DS), jnp.float32),
                pltpu.VMEM((RB, DS), jnp.float32),
                pltpu.VMEM((CHUNKS, CHUNK), jnp.int32),
                pltpu.VMEM((CHUNKS, CHUNK), jnp.int32),
                pltpu.VMEM((CHUNKS, CHUNK), jnp.float32),
            ] + [pltpu.SemaphoreType.DMA] * 9,
        ))
    return _SC_CALL_CACHE[0](*args)


def kernel(x_u, x_v, edge_u, edge_v, edge_val, edge_val_t, W):
    tabs = _project(x_u, x_v, W)
    pad = E_PAD - E

    def pad4(a):
        return jnp.pad(a, ((0, 0), (0, pad))).reshape(NSUP, N_TILES, CHUNKS, CHUNK)

    eu = pad4(edge_u)
    ev = pad4(edge_v)
    val = pad4(edge_val)
    valt = pad4(edge_val_t)
    out_u, out_v = _sc_call(*tabs, eu, ev, val, valt)
    return out_u[:N_NODES], out_v[:N_NODES]
